# baseline (speedup 1.0000x reference)
import jax
import jax.numpy as jnp
from jax.experimental import pallas as pl
from jax.experimental.pallas import tpu as pltpu

_LN_EPS = 1e-5


def _fused_mlp_kernel(n_in):
    def body(*refs):
        xs = refs[:n_in]
        w1s = refs[n_in:2 * n_in]
        b1, g, be, w2, b2 = refs[2 * n_in:2 * n_in + 5]
        out = refs[-1]

        h = b1[...].astype(jnp.float32)
        acc = None
        for x, w in zip(xs, w1s):
            d = jnp.dot(x[...].astype(jnp.bfloat16), w[...],
                        preferred_element_type=jnp.float32)
            acc = d if acc is None else acc + d
        h = acc + h
        mu = jnp.mean(h, axis=-1, keepdims=True)
        hc = h - mu
        var = jnp.mean(hc * hc, axis=-1, keepdims=True)
        h = hc * jax.lax.rsqrt(var + _LN_EPS) * g[...] + be[...]
        h = jnp.maximum(h, 0.0)
        out[...] = jnp.dot(h.astype(jnp.bfloat16), w2[...],
                           preferred_element_type=jnp.float32) + b2[...]

    return body


def _fused_mlp(xs, p, n_splits, block_rows):
    n = xs[0].shape[0]
    din_each = [x.shape[1] for x in xs]
    w1 = p["W1"]
    w1s = []
    off = 0
    for d in din_each:
        w1s.append(w1[off:off + d].astype(jnp.bfloat16))
        off += d
    w2 = p["W2"].astype(jnp.bfloat16)
    b1 = p["b1"].reshape(1, -1)
    g = p["g"].reshape(1, -1)
    be = p["be"].reshape(1, -1)
    b2 = p["b2"].reshape(1, -1)
    dout = w2.shape[1]
    dh = w2.shape[0]

    grid = (n // block_rows,)
    x_specs = [pl.BlockSpec((block_rows, d), lambda i: (i, 0)) for d in din_each]
    w_specs = [pl.BlockSpec((d, dh), lambda i: (0, 0)) for d in din_each]
    vec_spec = pl.BlockSpec((1, dh), lambda i: (0, 0))
    w2_spec = pl.BlockSpec((dh, dout), lambda i: (0, 0))
    b2_spec = pl.BlockSpec((1, dout), lambda i: (0, 0))
    out_spec = pl.BlockSpec((block_rows, dout), lambda i: (i, 0))

    return pl.pallas_call(
        _fused_mlp_kernel(len(xs)),
        grid=grid,
        in_specs=x_specs + w_specs + [vec_spec, vec_spec, vec_spec, w2_spec, b2_spec],
        out_specs=out_spec,
        out_shape=jax.ShapeDtypeStruct((n, dout), jnp.float32),
    )(*xs, *w1s, b1, g, be, w2, b2)


def kernel(nodes, edges, params, eidx):
    bus = nodes["bus"]

    s, r = eidx["ac"][0], eidx["ac"][1]
    ue_ac = _fused_mlp([edges["ac"], bus[s], bus[r]], params["e_ac"], 3, 2000)
    r_ac = r

    s, r = eidx["tr"][0], eidx["tr"][1]
    ue_tr = _fused_mlp([edges["tr"], bus[s], bus[r]], params["e_tr"], 3, 2000)
    r_tr = r

    s, r = eidx["gen"][0], eidx["gen"][1]
    ue_gen = _fused_mlp([bus[s], nodes["generator"][r]], params["e_gen"], 2, 1000)
    r_gen = r

    s, r = eidx["load"][0], eidx["load"][1]
    ue_load = _fused_mlp([bus[s], nodes["load"][r]], params["e_load"], 2, 2000)
    r_load = r

    s, r = eidx["shunt"][0], eidx["shunt"][1]
    ue_shunt = _fused_mlp([bus[s], nodes["shunt"][r]], params["e_shunt"], 2, 2000)
    r_shunt = r

    ed = ue_ac.shape[1]
    RNG = 12512
    parts = []
    for p in range(4):
        lo = p * RNG
        acc = jnp.zeros((RNG + 8, ed), jnp.float32)
        la = r_ac - lo
        la = jnp.where((la >= 0) & (la < RNG), la, RNG)
        acc = acc.at[la].add(ue_ac, mode="drop")
        lt = r_tr - lo
        lt = jnp.where((lt >= 0) & (lt < RNG), lt, RNG)
        acc = acc.at[lt].add(ue_tr, mode="drop")
        parts.append(acc[:RNG])
    msg_bus = jnp.concatenate(parts, axis=0)[:bus.shape[0]]
    msg_gen = jnp.zeros((nodes["generator"].shape[0], ed), jnp.float32).at[r_gen].add(ue_gen)
    msg_load = jnp.zeros((nodes["load"].shape[0], ed), jnp.float32).at[r_load].add(ue_load)
    msg_shunt = jnp.zeros((nodes["shunt"].shape[0], ed), jnp.float32).at[r_shunt].add(ue_shunt)

    nb = _fused_mlp([bus, msg_bus], params["n_bus"], 2, 2000)
    ng = _fused_mlp([nodes["generator"], msg_gen], params["n_generator"], 2, 1000)
    nl = _fused_mlp([nodes["load"], msg_load], params["n_load"], 2, 2000)
    ns = _fused_mlp([nodes["shunt"], msg_shunt], params["n_shunt"], 2, 2000)

    return (nb, ng, nl, ns, ue_ac, ue_tr, ue_gen, ue_load, ue_shunt)


# speedup vs baseline: 2.2985x; 2.2985x over previous
import functools

import jax
import jax.numpy as jnp
from jax import lax
from jax.experimental import pallas as pl
from jax.experimental.pallas import tpu as pltpu
from jax.experimental.pallas import tpu_sc as plsc

_LN_EPS = 1e-5

_NC, _NS = 2, 16
_W = 64

_BUS_RANGE = 12512
_BUS_ALLOC, _BUS_SPAN = 12544, 784
_GEN_ALLOC, _GEN_SPAN = 5120, 320
_LOAD_ALLOC, _LOAD_SPAN = 10240, 640
_SHUNT_ALLOC, _SHUNT_SPAN = 2048, 128
_ZROWS = 784


def _pad_rows(x, rows, fill):
    if x.shape[0] == rows:
        return x
    pad = jnp.full((rows - x.shape[0],) + x.shape[1:], fill, x.dtype)
    return jnp.concatenate([x, pad], axis=0)


def _stream_scatter_job(upd_hbm, idx_hbm, nwin, lo, acc, idx_v, loc_v, upd_v,
                        sems, s):
    jmax = -(-nwin // _NS)

    def issue(j, b):
        w = s + _NS * j

        @pl.when(w < nwin)
        def _():
            base = w * _W
            pltpu.async_copy(idx_hbm.at[pl.ds(base, _W)], idx_v.at[b], sems[b])
            pltpu.async_copy(upd_hbm.at[pl.ds(base, _W)], upd_v.at[b], sems[b])

    def process(j, b):
        w = s + _NS * j

        @pl.when(w < nwin)
        def _():
            pltpu.make_async_copy(idx_hbm.at[pl.ds(0, _W)], idx_v.at[b],
                                  sems[b]).wait()
            pltpu.make_async_copy(upd_hbm.at[pl.ds(0, _W)], upd_v.at[b],
                                  sems[b]).wait()
            if lo is None:
                pltpu.sync_copy(upd_v.at[b], acc.at[idx_v.at[b]], add=True)
            else:
                for k in range(_W // 16):
                    v = idx_v[b, pl.ds(16 * k, 16)]
                    u = v - lo
                    ok = (u >= 0) & (u < _BUS_RANGE)
                    loc_v[b, pl.ds(16 * k, 16)] = jnp.where(
                        ok, u, jnp.int32(_BUS_RANGE))
                pltpu.sync_copy(upd_v.at[b], acc.at[loc_v.at[b]], add=True)

    issue(0, 0)

    @pl.loop(0, -(-jmax // 2))
    def _(p):
        j0 = 2 * p
        issue(j0 + 1, 1)
        process(j0, 0)
        issue(j0 + 2, 0)
        process(j0 + 1, 1)


def _sc_scatter_all(ue_ac, r_ac, ue_tr, r_tr, ue_gen, r_gen,
                    ue_load, r_load, ue_shunt, r_shunt):
    zeros = jnp.zeros((_ZROWS, 128), jnp.float32)

    out_types = (
        jax.ShapeDtypeStruct((4, _BUS_ALLOC, 128), jnp.float32),
        jax.ShapeDtypeStruct((_GEN_ALLOC, 128), jnp.float32),
        jax.ShapeDtypeStruct((_LOAD_ALLOC, 128), jnp.float32),
        jax.ShapeDtypeStruct((_SHUNT_ALLOC, 128), jnp.float32),
    )

    mesh = plsc.VectorSubcoreMesh(core_axis_name="c", subcore_axis_name="s")

    @functools.partial(
        pl.kernel,
        out_type=out_types,
        mesh=mesh,
        scratch_types=[
            pltpu.VMEM_SHARED((_BUS_ALLOC, 128), jnp.float32),
            pltpu.VMEM((2, _W), jnp.int32),
            pltpu.VMEM((2, _W), jnp.int32),
            pltpu.VMEM((2, _W, 128), jnp.float32),
            pltpu.SemaphoreType.DMA,
            pltpu.SemaphoreType.DMA,
        ],
    )
    def scatter_kernel(ue_ac_h, rac_h, ue_tr_h, rtr_h, ue_g_h, rg_h,
                       ue_l_h, rl_h, ue_s_h, rs_h, z_h,
                       out_bus, out_gen, out_load, out_shunt,
                       acc, idx_v, loc_v, upd_v, sem0, sem1):
        c = lax.axis_index("c")
        s = lax.axis_index("s")
        sems = (sem0, sem1)
        bus_jobs = ((ue_ac_h, rac_h, 400000 // _W), (ue_tr_h, rtr_h, 50048 // _W))

        for p in range(2):
            rid = c * 2 + p
            lo = rid * _BUS_RANGE
            pltpu.sync_copy(z_h.at[pl.ds(0, _BUS_SPAN)],
                            acc.at[pl.ds(s * _BUS_SPAN, _BUS_SPAN)])
            plsc.subcore_barrier()
            for upd_hbm, idx_hbm, nwin in bus_jobs:
                _stream_scatter_job(upd_hbm, idx_hbm, nwin, lo, acc,
                                    idx_v, loc_v, upd_v, sems, s)
            plsc.subcore_barrier()
            pltpu.sync_copy(acc.at[pl.ds(s * _BUS_SPAN, _BUS_SPAN)],
                            out_bus.at[rid, pl.ds(s * _BUS_SPAN, _BUS_SPAN)])
            plsc.subcore_barrier()

        small = (
            (0, (ue_g_h, rg_h, 5120 // _W), _GEN_SPAN, out_gen),
            (1, (ue_l_h, rl_h, 10112 // _W), _LOAD_SPAN, out_load),
            (0, (ue_s_h, rs_h, 2048 // _W), _SHUNT_SPAN, out_shunt),
        )
        for owner, (upd_hbm, idx_hbm, nwin), span, out_ref in small:
            @pl.when(c == owner)
            def _(upd_hbm=upd_hbm, idx_hbm=idx_hbm, nwin=nwin, span=span,
                  out_ref=out_ref):
                pltpu.sync_copy(z_h.at[pl.ds(0, span)],
                                acc.at[pl.ds(s * span, span)])
                plsc.subcore_barrier()
                _stream_scatter_job(upd_hbm, idx_hbm, nwin, None, acc,
                                    idx_v, loc_v, upd_v, sems, s)
                plsc.subcore_barrier()
                pltpu.sync_copy(acc.at[pl.ds(s * span, span)],
                                out_ref.at[pl.ds(s * span, span)])
                plsc.subcore_barrier()

    return scatter_kernel(ue_ac, r_ac, ue_tr, r_tr, ue_gen, r_gen,
                          ue_load, r_load, ue_shunt, r_shunt, zeros)


def _fused_mlp_kernel(n_in):
    def body(*refs):
        xs = refs[:n_in]
        w1s = refs[n_in:2 * n_in]
        b1, g, be, w2, b2 = refs[2 * n_in:2 * n_in + 5]
        out = refs[-1]

        h = b1[...].astype(jnp.float32)
        acc = None
        for x, w in zip(xs, w1s):
            v = x[...]
            if v.ndim == 3:
                v = v[0]
            d = jnp.dot(v.astype(jnp.bfloat16), w[...],
                        preferred_element_type=jnp.float32)
            acc = d if acc is None else acc + d
        h = acc + h
        mu = jnp.mean(h, axis=-1, keepdims=True)
        hc = h - mu
        var = jnp.mean(hc * hc, axis=-1, keepdims=True)
        h = hc * jax.lax.rsqrt(var + _LN_EPS) * g[...] + be[...]
        h = jnp.maximum(h, 0.0)
        out[...] = jnp.dot(h.astype(jnp.bfloat16), w2[...],
                           preferred_element_type=jnp.float32) + b2[...]

    return body


def _fused_mlp(xs, p, n_splits, block_rows):
    n = xs[0].shape[0]
    din_each = [x.shape[1] for x in xs]
    w1 = p["W1"]
    w1s = []
    off = 0
    for d in din_each:
        w1s.append(w1[off:off + d].astype(jnp.bfloat16))
        off += d
    w2 = p["W2"].astype(jnp.bfloat16)
    b1 = p["b1"].reshape(1, -1)
    g = p["g"].reshape(1, -1)
    be = p["be"].reshape(1, -1)
    b2 = p["b2"].reshape(1, -1)
    dout = w2.shape[1]
    dh = w2.shape[0]

    grid = (n // block_rows,)
    x_specs = [pl.BlockSpec((block_rows, d), lambda i: (i, 0)) for d in din_each]
    w_specs = [pl.BlockSpec((d, dh), lambda i: (0, 0)) for d in din_each]
    vec_spec = pl.BlockSpec((1, dh), lambda i: (0, 0))
    w2_spec = pl.BlockSpec((dh, dout), lambda i: (0, 0))
    b2_spec = pl.BlockSpec((1, dout), lambda i: (0, 0))
    out_spec = pl.BlockSpec((block_rows, dout), lambda i: (i, 0))

    return pl.pallas_call(
        _fused_mlp_kernel(len(xs)),
        grid=grid,
        in_specs=x_specs + w_specs + [vec_spec, vec_spec, vec_spec, w2_spec, b2_spec],
        out_specs=out_spec,
        out_shape=jax.ShapeDtypeStruct((n, dout), jnp.float32),
    )(*xs, *w1s, b1, g, be, w2, b2)


def _node_mlp_bus(x_node, msg_parts, p):
    x_node = _pad_rows(x_node, 4 * _BUS_RANGE, 0.0)
    n, nd = x_node.shape
    block_rows = 3128
    w1 = p["W1"]
    w1s = [w1[0:nd].astype(jnp.bfloat16), w1[nd:].astype(jnp.bfloat16)]
    w2 = p["W2"].astype(jnp.bfloat16)
    b1 = p["b1"].reshape(1, -1)
    g = p["g"].reshape(1, -1)
    be = p["be"].reshape(1, -1)
    b2 = p["b2"].reshape(1, -1)
    dh = w2.shape[0]
    dout = w2.shape[1]
    per_part = _BUS_RANGE // block_rows

    grid = (n // block_rows,)
    x_specs = [
        pl.BlockSpec((block_rows, nd), lambda i: (i, 0)),
        pl.BlockSpec((1, block_rows, 128),
                     lambda i: (i // per_part, i % per_part, 0)),
    ]
    w_specs = [pl.BlockSpec((nd, dh), lambda i: (0, 0)),
               pl.BlockSpec((128, dh), lambda i: (0, 0))]
    vec_spec = pl.BlockSpec((1, dh), lambda i: (0, 0))
    w2_spec = pl.BlockSpec((dh, dout), lambda i: (0, 0))
    b2_spec = pl.BlockSpec((1, dout), lambda i: (0, 0))
    out_spec = pl.BlockSpec((block_rows, dout), lambda i: (i, 0))

    return pl.pallas_call(
        _fused_mlp_kernel(2),
        grid=grid,
        in_specs=x_specs + w_specs + [vec_spec, vec_spec, vec_spec, w2_spec, b2_spec],
        out_specs=out_spec,
        out_shape=jax.ShapeDtypeStruct((n, dout), jnp.float32),
    )(x_node, msg_parts, *w1s, b1, g, be, w2, b2)


def kernel(nodes, edges, params, eidx):
    bus = nodes["bus"]

    s, r = eidx["ac"][0], eidx["ac"][1]
    ue_ac = _fused_mlp([edges["ac"], bus[s], bus[r]], params["e_ac"], 3, 2000)
    r_ac = r

    s, r = eidx["tr"][0], eidx["tr"][1]
    ue_tr = _fused_mlp([edges["tr"], bus[s], bus[r]], params["e_tr"], 3, 2000)
    r_tr = r

    s, r = eidx["gen"][0], eidx["gen"][1]
    ue_gen = _fused_mlp([bus[s], nodes["generator"][r]], params["e_gen"], 2, 1000)
    r_gen = r

    s, r = eidx["load"][0], eidx["load"][1]
    ue_load = _fused_mlp([bus[s], nodes["load"][r]], params["e_load"], 2, 2000)
    r_load = r

    s, r = eidx["shunt"][0], eidx["shunt"][1]
    ue_shunt = _fused_mlp([bus[s], nodes["shunt"][r]], params["e_shunt"], 2, 2000)
    r_shunt = r

    ue_tr_p = _pad_rows(ue_tr, 50048, 0.0)
    r_tr_p = _pad_rows(r_tr, 50048, 50000)
    ue_gen_p = _pad_rows(ue_gen, 5120, 0.0)
    r_gen_p = _pad_rows(r_gen, 5120, 5000)
    ue_load_p = _pad_rows(ue_load, 10112, 0.0)
    r_load_p = _pad_rows(r_load, 10112, 10000)
    ue_shunt_p = _pad_rows(ue_shunt, 2048, 0.0)
    r_shunt_p = _pad_rows(r_shunt, 2048, 2000)

    msg_bus, msg_gen, msg_load, msg_shunt = _sc_scatter_all(
        ue_ac, r_ac, ue_tr_p, r_tr_p, ue_gen_p, r_gen_p,
        ue_load_p, r_load_p, ue_shunt_p, r_shunt_p)

    nb = _node_mlp_bus(bus, msg_bus, params["n_bus"])[:bus.shape[0]]
    ng = _fused_mlp([nodes["generator"], msg_gen], params["n_generator"], 2, 1000)
    nl = _fused_mlp([nodes["load"], msg_load], params["n_load"], 2, 2000)
    ns = _fused_mlp([nodes["shunt"], msg_shunt], params["n_shunt"], 2, 2000)

    return (nb, ng, nl, ns, ue_ac, ue_tr, ue_gen, ue_load, ue_shunt)


# speedup vs baseline: 3.2728x; 1.4239x over previous
import functools

import jax
import jax.numpy as jnp
from jax import lax
from jax.experimental import pallas as pl
from jax.experimental.pallas import tpu as pltpu
from jax.experimental.pallas import tpu_sc as plsc

_LN_EPS = 1e-5

_NC, _NS = 2, 16
_W = 64

_BUS_RANGE = 12512
_BUS_ALLOC, _BUS_SPAN = 12544, 784
_GEN_ALLOC, _GEN_SPAN = 5120, 320
_LOAD_ALLOC, _LOAD_SPAN = 10240, 640
_SHUNT_ALLOC, _SHUNT_SPAN = 2048, 128
_ZROWS = 784


def _pad_rows(x, rows, fill):
    if x.shape[0] == rows:
        return x
    pad = jnp.full((rows - x.shape[0],) + x.shape[1:], fill, x.dtype)
    return jnp.concatenate([x, pad], axis=0)


def _stream_scatter_job(upd_hbm, idx_hbm, nwin, lo, acc, idx_v, loc_v, upd_v,
                        sems, s):
    jmax = -(-nwin // _NS)

    def issue(j, b):
        w = s + _NS * j

        @pl.when(w < nwin)
        def _():
            base = w * _W
            pltpu.async_copy(idx_hbm.at[pl.ds(base, _W)], idx_v.at[b], sems[b])
            pltpu.async_copy(upd_hbm.at[pl.ds(base, _W)], upd_v.at[b], sems[b])

    def process(j, b):
        w = s + _NS * j

        @pl.when(w < nwin)
        def _():
            pltpu.make_async_copy(idx_hbm.at[pl.ds(0, _W)], idx_v.at[b],
                                  sems[b]).wait()
            pltpu.make_async_copy(upd_hbm.at[pl.ds(0, _W)], upd_v.at[b],
                                  sems[b]).wait()
            if lo is None:
                pltpu.sync_copy(upd_v.at[b], acc.at[idx_v.at[b]], add=True)
            else:
                for k in range(_W // 16):
                    v = idx_v[b, pl.ds(16 * k, 16)]
                    u = v - lo
                    ok = (u >= 0) & (u < _BUS_RANGE)
                    loc_v[b, pl.ds(16 * k, 16)] = jnp.where(
                        ok, u, jnp.int32(_BUS_RANGE))
                pltpu.sync_copy(upd_v.at[b], acc.at[loc_v.at[b]], add=True)

    issue(0, 0)

    @pl.loop(0, -(-jmax // 2))
    def _(p):
        j0 = 2 * p
        issue(j0 + 1, 1)
        process(j0, 0)
        issue(j0 + 2, 0)
        process(j0 + 1, 1)


def _sc_scatter_all(ue_ac, r_ac, ue_tr, r_tr, ue_gen, r_gen,
                    ue_load, r_load, ue_shunt, r_shunt):
    zeros = jnp.zeros((_ZROWS, 128), jnp.float32)

    out_types = (
        jax.ShapeDtypeStruct((4, _BUS_ALLOC, 128), jnp.float32),
        jax.ShapeDtypeStruct((_GEN_ALLOC, 128), jnp.float32),
        jax.ShapeDtypeStruct((_LOAD_ALLOC, 128), jnp.float32),
        jax.ShapeDtypeStruct((_SHUNT_ALLOC, 128), jnp.float32),
    )

    mesh = plsc.VectorSubcoreMesh(core_axis_name="c", subcore_axis_name="s")

    @functools.partial(
        pl.kernel,
        out_type=out_types,
        mesh=mesh,
        scratch_types=[
            pltpu.VMEM_SHARED((_BUS_ALLOC, 128), jnp.float32),
            pltpu.VMEM((2, _W), jnp.int32),
            pltpu.VMEM((2, _W), jnp.int32),
            pltpu.VMEM((2, _W, 128), jnp.float32),
            pltpu.SemaphoreType.DMA,
            pltpu.SemaphoreType.DMA,
        ],
    )
    def scatter_kernel(ue_ac_h, rac_h, ue_tr_h, rtr_h, ue_g_h, rg_h,
                       ue_l_h, rl_h, ue_s_h, rs_h, z_h,
                       out_bus, out_gen, out_load, out_shunt,
                       acc, idx_v, loc_v, upd_v, sem0, sem1):
        c = lax.axis_index("c")
        s = lax.axis_index("s")
        sems = (sem0, sem1)
        bus_jobs = ((ue_ac_h, rac_h, 400000 // _W), (ue_tr_h, rtr_h, 50048 // _W))

        for p in range(2):
            rid = c * 2 + p
            lo = rid * _BUS_RANGE
            pltpu.sync_copy(z_h.at[pl.ds(0, _BUS_SPAN)],
                            acc.at[pl.ds(s * _BUS_SPAN, _BUS_SPAN)])
            plsc.subcore_barrier()
            for upd_hbm, idx_hbm, nwin in bus_jobs:
                _stream_scatter_job(upd_hbm, idx_hbm, nwin, lo, acc,
                                    idx_v, loc_v, upd_v, sems, s)
            plsc.subcore_barrier()
            pltpu.sync_copy(acc.at[pl.ds(s * _BUS_SPAN, _BUS_SPAN)],
                            out_bus.at[rid, pl.ds(s * _BUS_SPAN, _BUS_SPAN)])
            plsc.subcore_barrier()

        small = (
            (0, (ue_g_h, rg_h, 5120 // _W), _GEN_SPAN, out_gen),
            (1, (ue_l_h, rl_h, 10112 // _W), _LOAD_SPAN, out_load),
            (0, (ue_s_h, rs_h, 2048 // _W), _SHUNT_SPAN, out_shunt),
        )
        for owner, (upd_hbm, idx_hbm, nwin), span, out_ref in small:
            @pl.when(c == owner)
            def _(upd_hbm=upd_hbm, idx_hbm=idx_hbm, nwin=nwin, span=span,
                  out_ref=out_ref):
                pltpu.sync_copy(z_h.at[pl.ds(0, span)],
                                acc.at[pl.ds(s * span, span)])
                plsc.subcore_barrier()
                _stream_scatter_job(upd_hbm, idx_hbm, nwin, None, acc,
                                    idx_v, loc_v, upd_v, sems, s)
                plsc.subcore_barrier()
                pltpu.sync_copy(acc.at[pl.ds(s * span, span)],
                                out_ref.at[pl.ds(s * span, span)])
                plsc.subcore_barrier()

    return scatter_kernel(ue_ac, r_ac, ue_tr, r_tr, ue_gen, r_gen,
                          ue_load, r_load, ue_shunt, r_shunt, zeros)


def _gather_job(idx_hbm, tab_hbm, out_hbm, nwin, wid, idx_v, rows_v, sems):
    semi, semo = sems
    jmax = -(-nwin // (_NC * _NS))

    def issue_idx(j, b):
        w = wid + _NC * _NS * j

        @pl.when(w < nwin)
        def _():
            pltpu.async_copy(idx_hbm.at[pl.ds(w * _W, _W)], idx_v.at[b],
                             semi[b])

    def process(j, b):
        w = wid + _NC * _NS * j
        stride2 = 2 * _NC * _NS

        @pl.when((w >= stride2) & (w - stride2 < nwin))
        def _():
            pltpu.make_async_copy(rows_v.at[b], out_hbm.at[pl.ds(0, _W)],
                                  semo[b]).wait()

        @pl.when(w < nwin)
        def _():
            pltpu.make_async_copy(idx_hbm.at[pl.ds(0, _W)], idx_v.at[b],
                                  semi[b]).wait()
            pltpu.async_copy(tab_hbm.at[idx_v.at[b]], rows_v.at[b],
                             semo[b]).wait()
            pltpu.async_copy(rows_v.at[b], out_hbm.at[pl.ds(w * _W, _W)],
                             semo[b])

    issue_idx(0, 0)

    @pl.loop(0, -(-jmax // 2) + 1)
    def _(p):
        j0 = 2 * p
        issue_idx(j0 + 1, 1)
        process(j0, 0)
        issue_idx(j0 + 2, 0)
        process(j0 + 1, 1)


def _sc_gather_all(jobs):
    out_types = tuple(jax.ShapeDtypeStruct((n, 128), jnp.float32)
                      for _, _, n in jobs)
    mesh = plsc.VectorSubcoreMesh(core_axis_name="c", subcore_axis_name="s")
    njobs = len(jobs)

    @functools.partial(
        pl.kernel,
        out_type=out_types,
        mesh=mesh,
        scratch_types=[
            pltpu.VMEM((2, _W), jnp.int32),
            pltpu.VMEM((2, _W, 128), jnp.float32),
            pltpu.SemaphoreType.DMA,
            pltpu.SemaphoreType.DMA,
            pltpu.SemaphoreType.DMA,
            pltpu.SemaphoreType.DMA,
        ],
    )
    def gather_kernel(*refs):
        idx_refs = refs[0:2 * njobs:2]
        tab_refs = refs[1:2 * njobs:2]
        out_refs = refs[2 * njobs:3 * njobs]
        idx_v, rows_v, si0, si1, so0, so1 = refs[3 * njobs:]
        c = lax.axis_index("c")
        s = lax.axis_index("s")
        wid = s * _NC + c
        for q in range(njobs):
            nwin = jobs[q][2] // _W
            _gather_job(idx_refs[q], tab_refs[q], out_refs[q], nwin, wid,
                        idx_v, rows_v, ((si0, si1), (so0, so1)))

    args = []
    for idx, tab, _ in jobs:
        args += [idx, tab]
    return gather_kernel(*args)


def _fused_mlp_kernel(n_in):
    def body(*refs):
        xs = refs[:n_in]
        w1s = refs[n_in:2 * n_in]
        b1, g, be, w2, b2 = refs[2 * n_in:2 * n_in + 5]
        out = refs[-1]

        h = b1[...].astype(jnp.float32)
        acc = None
        for x, w in zip(xs, w1s):
            v = x[...]
            if v.ndim == 3:
                v = v[0]
            d = jnp.dot(v.astype(jnp.bfloat16), w[...],
                        preferred_element_type=jnp.float32)
            acc = d if acc is None else acc + d
        h = acc + h
        mu = jnp.mean(h, axis=-1, keepdims=True)
        hc = h - mu
        var = jnp.mean(hc * hc, axis=-1, keepdims=True)
        h = hc * jax.lax.rsqrt(var + _LN_EPS) * g[...] + be[...]
        h = jnp.maximum(h, 0.0)
        out[...] = jnp.dot(h.astype(jnp.bfloat16), w2[...],
                           preferred_element_type=jnp.float32) + b2[...]

    return body


def _fused_mlp(xs, p, n_splits, block_rows):
    n = xs[0].shape[0]
    din_each = [x.shape[1] for x in xs]
    w1 = p["W1"]
    w1s = []
    off = 0
    for d in din_each:
        w1s.append(w1[off:off + d].astype(jnp.bfloat16))
        off += d
    w2 = p["W2"].astype(jnp.bfloat16)
    b1 = p["b1"].reshape(1, -1)
    g = p["g"].reshape(1, -1)
    be = p["be"].reshape(1, -1)
    b2 = p["b2"].reshape(1, -1)
    dout = w2.shape[1]
    dh = w2.shape[0]

    grid = (n // block_rows,)
    x_specs = [pl.BlockSpec((block_rows, d), lambda i: (i, 0)) for d in din_each]
    w_specs = [pl.BlockSpec((d, dh), lambda i: (0, 0)) for d in din_each]
    vec_spec = pl.BlockSpec((1, dh), lambda i: (0, 0))
    w2_spec = pl.BlockSpec((dh, dout), lambda i: (0, 0))
    b2_spec = pl.BlockSpec((1, dout), lambda i: (0, 0))
    out_spec = pl.BlockSpec((block_rows, dout), lambda i: (i, 0))

    return pl.pallas_call(
        _fused_mlp_kernel(len(xs)),
        grid=grid,
        in_specs=x_specs + w_specs + [vec_spec, vec_spec, vec_spec, w2_spec, b2_spec],
        out_specs=out_spec,
        out_shape=jax.ShapeDtypeStruct((n, dout), jnp.float32),
    )(*xs, *w1s, b1, g, be, w2, b2)


def _node_mlp_bus(x_node, msg_parts, p):
    x_node = _pad_rows(x_node, 4 * _BUS_RANGE, 0.0)
    n, nd = x_node.shape
    block_rows = 3128
    w1 = p["W1"]
    w1s = [w1[0:nd].astype(jnp.bfloat16), w1[nd:].astype(jnp.bfloat16)]
    w2 = p["W2"].astype(jnp.bfloat16)
    b1 = p["b1"].reshape(1, -1)
    g = p["g"].reshape(1, -1)
    be = p["be"].reshape(1, -1)
    b2 = p["b2"].reshape(1, -1)
    dh = w2.shape[0]
    dout = w2.shape[1]
    per_part = _BUS_RANGE // block_rows

    grid = (n // block_rows,)
    x_specs = [
        pl.BlockSpec((block_rows, nd), lambda i: (i, 0)),
        pl.BlockSpec((1, block_rows, 128),
                     lambda i: (i // per_part, i % per_part, 0)),
    ]
    w_specs = [pl.BlockSpec((nd, dh), lambda i: (0, 0)),
               pl.BlockSpec((128, dh), lambda i: (0, 0))]
    vec_spec = pl.BlockSpec((1, dh), lambda i: (0, 0))
    w2_spec = pl.BlockSpec((dh, dout), lambda i: (0, 0))
    b2_spec = pl.BlockSpec((1, dout), lambda i: (0, 0))
    out_spec = pl.BlockSpec((block_rows, dout), lambda i: (i, 0))

    return pl.pallas_call(
        _fused_mlp_kernel(2),
        grid=grid,
        in_specs=x_specs + w_specs + [vec_spec, vec_spec, vec_spec, w2_spec, b2_spec],
        out_specs=out_spec,
        out_shape=jax.ShapeDtypeStruct((n, dout), jnp.float32),
    )(x_node, msg_parts, *w1s, b1, g, be, w2, b2)


def kernel(nodes, edges, params, eidx):
    bus = nodes["bus"]

    s_ac, r_ac = eidx["ac"][0], eidx["ac"][1]
    s_tr, r_tr = eidx["tr"][0], eidx["tr"][1]
    s_gen, r_gen = eidx["gen"][0], eidx["gen"][1]
    s_load, r_load = eidx["load"][0], eidx["load"][1]
    s_shunt, r_shunt = eidx["shunt"][0], eidx["shunt"][1]

    gjobs = (
        (s_ac, bus, 400000), (r_ac, bus, 400000),
        (_pad_rows(s_tr, 50048, 0), bus, 50048),
        (_pad_rows(r_tr, 50048, 0), bus, 50048),
        (_pad_rows(s_gen, 5120, 0), bus, 5120),
        (_pad_rows(r_gen, 5120, 0), nodes["generator"], 5120),
        (_pad_rows(s_load, 10112, 0), bus, 10112),
        (_pad_rows(r_load, 10112, 0), nodes["load"], 10112),
        (_pad_rows(s_shunt, 2048, 0), bus, 2048),
        (_pad_rows(r_shunt, 2048, 0), nodes["shunt"], 2048),
    )
    (g_ac_s, g_ac_r, g_tr_s, g_tr_r, g_gen_s, g_gen_r,
     g_load_s, g_load_r, g_shunt_s, g_shunt_r) = _sc_gather_all(gjobs)

    ue_ac = _fused_mlp([edges["ac"], g_ac_s, g_ac_r], params["e_ac"], 3, 2000)
    ue_tr = _fused_mlp([edges["tr"], g_tr_s, g_tr_r], params["e_tr"], 3, 2000)
    ue_gen_f = _fused_mlp([g_gen_s, g_gen_r], params["e_gen"], 2, 1024)
    ue_load_f = _fused_mlp([g_load_s, g_load_r], params["e_load"], 2, 1264)
    ue_shunt_f = _fused_mlp([g_shunt_s, g_shunt_r], params["e_shunt"], 2, 2048)
    ue_gen = ue_gen_f[:s_gen.shape[0]]
    ue_load = ue_load_f[:s_load.shape[0]]
    ue_shunt = ue_shunt_f[:s_shunt.shape[0]]

    ue_tr_p = _pad_rows(ue_tr, 50048, 0.0)
    r_tr_p = _pad_rows(r_tr, 50048, 50000)
    r_gen_p = _pad_rows(r_gen, 5120, 5000)
    r_load_p = _pad_rows(r_load, 10112, 10000)
    r_shunt_p = _pad_rows(r_shunt, 2048, 2000)

    msg_bus, msg_gen, msg_load, msg_shunt = _sc_scatter_all(
        ue_ac, r_ac, ue_tr_p, r_tr_p, ue_gen_f, r_gen_p,
        ue_load_f, r_load_p, ue_shunt_f, r_shunt_p)

    nb = _node_mlp_bus(bus, msg_bus, params["n_bus"])[:bus.shape[0]]
    ng = _fused_mlp([nodes["generator"], msg_gen], params["n_generator"], 2, 1000)
    nl = _fused_mlp([nodes["load"], msg_load], params["n_load"], 2, 2000)
    ns = _fused_mlp([nodes["shunt"], msg_shunt], params["n_shunt"], 2, 2000)

    return (nb, ng, nl, ns, ue_ac, ue_tr, ue_gen, ue_load, ue_shunt)


# speedup vs baseline: 3.6052x; 1.1016x over previous
import functools

import jax
import jax.numpy as jnp
from jax import lax
from jax.experimental import pallas as pl
from jax.experimental.pallas import tpu as pltpu
from jax.experimental.pallas import tpu_sc as plsc

_LN_EPS = 1e-5

_NC, _NS = 2, 16
_W = 64

_BUS_RANGE = 12512
_BUS_ALLOC, _BUS_SPAN = 12544, 784
_GEN_ALLOC, _GEN_SPAN = 5120, 320
_LOAD_ALLOC, _LOAD_SPAN = 10240, 640
_SHUNT_ALLOC, _SHUNT_SPAN = 2048, 128
_ZROWS = 784


def _pad_rows(x, rows, fill):
    if x.shape[0] == rows:
        return x
    pad = jnp.full((rows - x.shape[0],) + x.shape[1:], fill, x.dtype)
    return jnp.concatenate([x, pad], axis=0)


def _stream_scatter_job(upd_hbm, idx_hbm, nwin, lo, acc, idx_v, loc_v, upd_v,
                        sems, s):
    jmax = -(-nwin // _NS)

    def issue(j, b):
        w = s + _NS * j

        @pl.when(w < nwin)
        def _():
            base = w * _W
            pltpu.async_copy(idx_hbm.at[pl.ds(base, _W)], idx_v.at[b], sems[b])
            pltpu.async_copy(upd_hbm.at[pl.ds(base, _W)], upd_v.at[b], sems[b])

    def process(j, b):
        w = s + _NS * j

        @pl.when(w < nwin)
        def _():
            pltpu.make_async_copy(idx_hbm.at[pl.ds(0, _W)], idx_v.at[b],
                                  sems[b]).wait()
            pltpu.make_async_copy(upd_hbm.at[pl.ds(0, _W)], upd_v.at[b],
                                  sems[b]).wait()
            if lo is None:
                pltpu.sync_copy(upd_v.at[b], acc.at[idx_v.at[b]], add=True)
            else:
                for k in range(_W // 16):
                    v = idx_v[b, pl.ds(16 * k, 16)]
                    u = v - lo
                    ok = (u >= 0) & (u < _BUS_RANGE)
                    loc_v[b, pl.ds(16 * k, 16)] = jnp.where(
                        ok, u, jnp.int32(_BUS_RANGE))
                pltpu.sync_copy(upd_v.at[b], acc.at[loc_v.at[b]], add=True)

    issue(0, 0)

    @pl.loop(0, -(-jmax // 2))
    def _(p):
        j0 = 2 * p
        issue(j0 + 1, 1)
        process(j0, 0)
        issue(j0 + 2, 0)
        process(j0 + 1, 1)


def _sc_scatter_all(ue_ac, r_ac, ue_tr, r_tr, ue_gen, r_gen,
                    ue_load, r_load, ue_shunt, r_shunt):
    zeros = jnp.zeros((_ZROWS, 128), jnp.float32)

    out_types = (
        jax.ShapeDtypeStruct((4, _BUS_ALLOC, 128), jnp.float32),
        jax.ShapeDtypeStruct((_GEN_ALLOC, 128), jnp.float32),
        jax.ShapeDtypeStruct((_LOAD_ALLOC, 128), jnp.float32),
        jax.ShapeDtypeStruct((_SHUNT_ALLOC, 128), jnp.float32),
    )

    mesh = plsc.VectorSubcoreMesh(core_axis_name="c", subcore_axis_name="s")

    @functools.partial(
        pl.kernel,
        out_type=out_types,
        mesh=mesh,
        scratch_types=[
            pltpu.VMEM_SHARED((_BUS_ALLOC, 128), jnp.float32),
            pltpu.VMEM((2, _W), jnp.int32),
            pltpu.VMEM((2, _W), jnp.int32),
            pltpu.VMEM((2, _W, 128), jnp.float32),
            pltpu.SemaphoreType.DMA,
            pltpu.SemaphoreType.DMA,
        ],
    )
    def scatter_kernel(ue_ac_h, rac_h, ue_tr_h, rtr_h, ue_g_h, rg_h,
                       ue_l_h, rl_h, ue_s_h, rs_h, z_h,
                       out_bus, out_gen, out_load, out_shunt,
                       acc, idx_v, loc_v, upd_v, sem0, sem1):
        c = lax.axis_index("c")
        s = lax.axis_index("s")
        sems = (sem0, sem1)
        bus_jobs = ((ue_ac_h, rac_h, 400000 // _W), (ue_tr_h, rtr_h, 50048 // _W))

        for p in range(2):
            rid = c * 2 + p
            lo = rid * _BUS_RANGE
            pltpu.sync_copy(z_h.at[pl.ds(0, _BUS_SPAN)],
                            acc.at[pl.ds(s * _BUS_SPAN, _BUS_SPAN)])
            plsc.subcore_barrier()
            for upd_hbm, idx_hbm, nwin in bus_jobs:
                _stream_scatter_job(upd_hbm, idx_hbm, nwin, lo, acc,
                                    idx_v, loc_v, upd_v, sems, s)
            plsc.subcore_barrier()
            pltpu.sync_copy(acc.at[pl.ds(s * _BUS_SPAN, _BUS_SPAN)],
                            out_bus.at[rid, pl.ds(s * _BUS_SPAN, _BUS_SPAN)])
            plsc.subcore_barrier()

        small = (
            (0, (ue_g_h, rg_h, 5120 // _W), _GEN_SPAN, out_gen),
            (1, (ue_l_h, rl_h, 10112 // _W), _LOAD_SPAN, out_load),
            (0, (ue_s_h, rs_h, 2048 // _W), _SHUNT_SPAN, out_shunt),
        )
        for owner, (upd_hbm, idx_hbm, nwin), span, out_ref in small:
            @pl.when(c == owner)
            def _(upd_hbm=upd_hbm, idx_hbm=idx_hbm, nwin=nwin, span=span,
                  out_ref=out_ref):
                pltpu.sync_copy(z_h.at[pl.ds(0, span)],
                                acc.at[pl.ds(s * span, span)])
                plsc.subcore_barrier()
                _stream_scatter_job(upd_hbm, idx_hbm, nwin, None, acc,
                                    idx_v, loc_v, upd_v, sems, s)
                plsc.subcore_barrier()
                pltpu.sync_copy(acc.at[pl.ds(s * span, span)],
                                out_ref.at[pl.ds(s * span, span)])
                plsc.subcore_barrier()

    return scatter_kernel(ue_ac, r_ac, ue_tr, r_tr, ue_gen, r_gen,
                          ue_load, r_load, ue_shunt, r_shunt, zeros)


_GW = 128


def _gather_job(idx_hbm, tab_hbm, out_hbm, nwin, wid, idx_v, rows_v, sems):
    semi, semo = sems
    jmax = -(-nwin // (_NC * _NS))

    def issue_idx(j, b):
        w = wid + _NC * _NS * j

        @pl.when(w < nwin)
        def _():
            pltpu.async_copy(idx_hbm.at[pl.ds(w * _GW, _GW)], idx_v.at[b],
                             semi[b])

    def process(j, b):
        w = wid + _NC * _NS * j
        stride2 = 2 * _NC * _NS

        @pl.when((w >= stride2) & (w - stride2 < nwin))
        def _():
            pltpu.make_async_copy(rows_v.at[b], out_hbm.at[pl.ds(0, _GW)],
                                  semo[b]).wait()

        @pl.when(w < nwin)
        def _():
            pltpu.make_async_copy(idx_hbm.at[pl.ds(0, _GW)], idx_v.at[b],
                                  semi[b]).wait()
            pltpu.async_copy(tab_hbm.at[idx_v.at[b]], rows_v.at[b],
                             semo[b]).wait()
            pltpu.async_copy(rows_v.at[b], out_hbm.at[pl.ds(w * _GW, _GW)],
                             semo[b])

    issue_idx(0, 0)

    @pl.loop(0, -(-jmax // 2) + 1)
    def _(p):
        j0 = 2 * p
        issue_idx(j0 + 1, 1)
        process(j0, 0)
        issue_idx(j0 + 2, 0)
        process(j0 + 1, 1)


def _sc_gather_all(jobs):
    out_types = tuple(jax.ShapeDtypeStruct((n, 128), jnp.float32)
                      for _, _, n in jobs)
    mesh = plsc.VectorSubcoreMesh(core_axis_name="c", subcore_axis_name="s")
    njobs = len(jobs)

    @functools.partial(
        pl.kernel,
        out_type=out_types,
        mesh=mesh,
        scratch_types=[
            pltpu.VMEM((2, _GW), jnp.int32),
            pltpu.VMEM((2, _GW, 128), jnp.float32),
            pltpu.SemaphoreType.DMA,
            pltpu.SemaphoreType.DMA,
            pltpu.SemaphoreType.DMA,
            pltpu.SemaphoreType.DMA,
        ],
    )
    def gather_kernel(*refs):
        idx_refs = refs[0:2 * njobs:2]
        tab_refs = refs[1:2 * njobs:2]
        out_refs = refs[2 * njobs:3 * njobs]
        idx_v, rows_v, si0, si1, so0, so1 = refs[3 * njobs:]
        c = lax.axis_index("c")
        s = lax.axis_index("s")
        wid = s * _NC + c
        for q in range(njobs):
            nwin = jobs[q][2] // _GW
            _gather_job(idx_refs[q], tab_refs[q], out_refs[q], nwin, wid,
                        idx_v, rows_v, ((si0, si1), (so0, so1)))

    args = []
    for idx, tab, _ in jobs:
        args += [idx, tab]
    return gather_kernel(*args)


def _fused_mlp_kernel(n_in):
    def body(*refs):
        xs = refs[:n_in]
        w1s = refs[n_in:2 * n_in]
        b1, g, be, w2, b2 = refs[2 * n_in:2 * n_in + 5]
        out = refs[-1]

        h = b1[...].astype(jnp.float32)
        acc = None
        for x, w in zip(xs, w1s):
            v = x[...]
            if v.ndim == 3:
                v = v[0]
            d = jnp.dot(v.astype(jnp.bfloat16), w[...],
                        preferred_element_type=jnp.float32)
            acc = d if acc is None else acc + d
        h = acc + h
        mu = jnp.mean(h, axis=-1, keepdims=True)
        hc = h - mu
        var = jnp.mean(hc * hc, axis=-1, keepdims=True)
        h = hc * jax.lax.rsqrt(var + _LN_EPS) * g[...] + be[...]
        h = jnp.maximum(h, 0.0)
        out[...] = jnp.dot(h.astype(jnp.bfloat16), w2[...],
                           preferred_element_type=jnp.float32) + b2[...]

    return body


def _fused_mlp(xs, p, n_splits, block_rows):
    n = xs[0].shape[0]
    din_each = [x.shape[1] for x in xs]
    w1 = p["W1"]
    w1s = []
    off = 0
    for d in din_each:
        w1s.append(w1[off:off + d].astype(jnp.bfloat16))
        off += d
    w2 = p["W2"].astype(jnp.bfloat16)
    b1 = p["b1"].reshape(1, -1)
    g = p["g"].reshape(1, -1)
    be = p["be"].reshape(1, -1)
    b2 = p["b2"].reshape(1, -1)
    dout = w2.shape[1]
    dh = w2.shape[0]

    grid = (n // block_rows,)
    x_specs = [pl.BlockSpec((block_rows, d), lambda i: (i, 0)) for d in din_each]
    w_specs = [pl.BlockSpec((d, dh), lambda i: (0, 0)) for d in din_each]
    vec_spec = pl.BlockSpec((1, dh), lambda i: (0, 0))
    w2_spec = pl.BlockSpec((dh, dout), lambda i: (0, 0))
    b2_spec = pl.BlockSpec((1, dout), lambda i: (0, 0))
    out_spec = pl.BlockSpec((block_rows, dout), lambda i: (i, 0))

    return pl.pallas_call(
        _fused_mlp_kernel(len(xs)),
        grid=grid,
        in_specs=x_specs + w_specs + [vec_spec, vec_spec, vec_spec, w2_spec, b2_spec],
        out_specs=out_spec,
        out_shape=jax.ShapeDtypeStruct((n, dout), jnp.float32),
    )(*xs, *w1s, b1, g, be, w2, b2)


def _node_mlp_bus(x_node, msg_parts, p):
    x_node = _pad_rows(x_node, 4 * _BUS_RANGE, 0.0)
    n, nd = x_node.shape
    block_rows = 3128
    w1 = p["W1"]
    w1s = [w1[0:nd].astype(jnp.bfloat16), w1[nd:].astype(jnp.bfloat16)]
    w2 = p["W2"].astype(jnp.bfloat16)
    b1 = p["b1"].reshape(1, -1)
    g = p["g"].reshape(1, -1)
    be = p["be"].reshape(1, -1)
    b2 = p["b2"].reshape(1, -1)
    dh = w2.shape[0]
    dout = w2.shape[1]
    per_part = _BUS_RANGE // block_rows

    grid = (n // block_rows,)
    x_specs = [
        pl.BlockSpec((block_rows, nd), lambda i: (i, 0)),
        pl.BlockSpec((1, block_rows, 128),
                     lambda i: (i // per_part, i % per_part, 0)),
    ]
    w_specs = [pl.BlockSpec((nd, dh), lambda i: (0, 0)),
               pl.BlockSpec((128, dh), lambda i: (0, 0))]
    vec_spec = pl.BlockSpec((1, dh), lambda i: (0, 0))
    w2_spec = pl.BlockSpec((dh, dout), lambda i: (0, 0))
    b2_spec = pl.BlockSpec((1, dout), lambda i: (0, 0))
    out_spec = pl.BlockSpec((block_rows, dout), lambda i: (i, 0))

    return pl.pallas_call(
        _fused_mlp_kernel(2),
        grid=grid,
        in_specs=x_specs + w_specs + [vec_spec, vec_spec, vec_spec, w2_spec, b2_spec],
        out_specs=out_spec,
        out_shape=jax.ShapeDtypeStruct((n, dout), jnp.float32),
    )(x_node, msg_parts, *w1s, b1, g, be, w2, b2)


def kernel(nodes, edges, params, eidx):
    bus = nodes["bus"]

    s_ac, r_ac = eidx["ac"][0], eidx["ac"][1]
    s_tr, r_tr = eidx["tr"][0], eidx["tr"][1]
    s_gen, r_gen = eidx["gen"][0], eidx["gen"][1]
    s_load, r_load = eidx["load"][0], eidx["load"][1]
    s_shunt, r_shunt = eidx["shunt"][0], eidx["shunt"][1]

    g_ac_s, g_ac_r = _sc_gather_all(
        ((s_ac, bus, 400000), (r_ac, bus, 400000)))
    gjobs = (
        (_pad_rows(s_tr, 50048, 0), bus, 50048),
        (_pad_rows(r_tr, 50048, 0), bus, 50048),
        (_pad_rows(s_gen, 5120, 0), bus, 5120),
        (_pad_rows(r_gen, 5120, 0), nodes["generator"], 5120),
        (_pad_rows(s_load, 10112, 0), bus, 10112),
        (_pad_rows(r_load, 10112, 0), nodes["load"], 10112),
        (_pad_rows(s_shunt, 2048, 0), bus, 2048),
        (_pad_rows(r_shunt, 2048, 0), nodes["shunt"], 2048),
    )
    (g_tr_s, g_tr_r, g_gen_s, g_gen_r,
     g_load_s, g_load_r, g_shunt_s, g_shunt_r) = _sc_gather_all(gjobs)

    ue_ac = _fused_mlp([edges["ac"], g_ac_s, g_ac_r], params["e_ac"], 3, 2000)
    ue_tr = _fused_mlp([edges["tr"], g_tr_s, g_tr_r], params["e_tr"], 3, 2000)
    ue_gen_f = _fused_mlp([g_gen_s, g_gen_r], params["e_gen"], 2, 1024)
    ue_load_f = _fused_mlp([g_load_s, g_load_r], params["e_load"], 2, 1264)
    ue_shunt_f = _fused_mlp([g_shunt_s, g_shunt_r], params["e_shunt"], 2, 2048)
    ue_gen = ue_gen_f[:s_gen.shape[0]]
    ue_load = ue_load_f[:s_load.shape[0]]
    ue_shunt = ue_shunt_f[:s_shunt.shape[0]]

    ue_tr_p = _pad_rows(ue_tr, 50048, 0.0)
    r_tr_p = _pad_rows(r_tr, 50048, 50000)
    r_gen_p = _pad_rows(r_gen, 5120, 5000)
    r_load_p = _pad_rows(r_load, 10112, 10000)
    r_shunt_p = _pad_rows(r_shunt, 2048, 2000)

    msg_bus, msg_gen, msg_load, msg_shunt = _sc_scatter_all(
        ue_ac, r_ac, ue_tr_p, r_tr_p, ue_gen_f, r_gen_p,
        ue_load_f, r_load_p, ue_shunt_f, r_shunt_p)

    nb = _node_mlp_bus(bus, msg_bus, params["n_bus"])[:bus.shape[0]]
    ng = _fused_mlp([nodes["generator"], msg_gen], params["n_generator"], 2, 1000)
    nl = _fused_mlp([nodes["load"], msg_load], params["n_load"], 2, 2000)
    ns = _fused_mlp([nodes["shunt"], msg_shunt], params["n_shunt"], 2, 2000)

    return (nb, ng, nl, ns, ue_ac, ue_tr, ue_gen, ue_load, ue_shunt)
